# baseline (speedup 1.0000x reference)
import functools
import math

import jax
import jax.numpy as jnp
from jax import lax
from jax.experimental import pallas as pl
from jax.experimental.pallas import tpu as pltpu
from jax.experimental.pallas import tpu_sc as plsc


def kernel(x, table):
    B0, S = x.shape
    V, D = table.shape
    W = 2 * D
    scale = math.sqrt(D)

    info = plsc.get_sparse_core_info()
    NC, NS, L = info.num_cores, info.num_subcores, info.num_lanes
    NW = NC * NS

    VB = 128
    n_full = V // VB
    tail = V - n_full * VB
    per_w = n_full // NW
    n_extra = n_full - per_w * NW

    mesh = plsc.VectorSubcoreMesh(core_axis_name="c", subcore_axis_name="s")
    cparams = pltpu.CompilerParams(
        use_tc_tiling_on_sc=True, needs_layout_passes=False
    )

    @functools.partial(
        pl.kernel,
        mesh=mesh,
        out_type=jax.ShapeDtypeStruct((V // 2, W), jnp.float32),
        scratch_types=[
            pltpu.VMEM((D, VB), jnp.float32),
            pltpu.VMEM((VB // 2, W), jnp.float32),
            pltpu.VMEM((D, D), jnp.float32),
        ],
        compiler_params=cparams,
    )
    def pack(tt_hbm, out_hbm, src_v, tr_v, tail_v):
        wid = lax.axis_index("s") * NC + lax.axis_index("c")
        lane = lax.iota(jnp.int32, L)

        rowsel = [dc * L + lane for dc in range(D // L)]

        def do_block(blk):
            v0 = blk * VB
            pltpu.sync_copy(tt_hbm.at[:, pl.ds(v0, VB)], src_v)
            for vl in range(VB):
                for dc in range(D // L):
                    vals = plsc.load_gather(
                        src_v, [rowsel[dc], jnp.full((L,), vl, jnp.int32)]
                    )
                    tr_v[vl // 2, pl.ds((vl % 2) * D + dc * L, L)] = vals * scale
            pltpu.sync_copy(tr_v, out_hbm.at[pl.ds(blk * (VB // 2), VB // 2)])

        def blk_body(c, carry):
            do_block(wid * per_w + c)
            return carry

        lax.fori_loop(0, per_w, blk_body, 0)

        @pl.when(wid < n_extra)
        def _():
            do_block(NW * per_w + wid)

        @pl.when(wid == n_extra)
        def _():
            v0 = n_full * VB

            def d_body(d, carry):
                pltpu.sync_copy(tt_hbm.at[d, pl.ds(v0, tail)], tail_v.at[d])
                return carry

            lax.fori_loop(0, D, d_body, 0)

            for vl in range(tail):
                for dc in range(D // L):
                    vals = plsc.load_gather(
                        tail_v, [rowsel[dc], jnp.full((L,), vl, jnp.int32)]
                    )
                    tr_v[vl // 2, pl.ds((vl % 2) * D + dc * L, L)] = vals * scale
            pltpu.sync_copy(
                tr_v.at[pl.ds(0, tail // 2)],
                out_hbm.at[pl.ds(v0 // 2, tail // 2)],
            )

    SB = 4
    n_sb = S // SB
    BW = B0 // NW
    RG = SB * BW

    @functools.partial(
        pl.kernel,
        mesh=mesh,
        out_type=jax.ShapeDtypeStruct((S, D, B0), jnp.float32),
        scratch_types=[
            pltpu.VMEM((SB, BW), jnp.int32),
            pltpu.VMEM((SB, BW), jnp.int32),
            pltpu.VMEM((RG, W), jnp.float32),
            pltpu.VMEM((SB, D, BW), jnp.float32),
            pltpu.SemaphoreType.DMA,
        ],
        compiler_params=cparams,
    )
    def emb(tp_hbm, idxh_hbm, par_hbm, out_hbm, idx_v, par_v, g_v, t_v, sem):
        wid = lax.axis_index("s") * NC + lax.axis_index("c")
        b0 = wid * BW
        lane = lax.iota(jnp.int32, L)

        def sb_body(sb, carry):
            pltpu.sync_copy(idxh_hbm.at[sb, :, pl.ds(b0, BW)], idx_v)
            pltpu.sync_copy(par_hbm.at[sb, :, pl.ds(b0, BW)], par_v)
            for sl in range(SB):
                pltpu.async_copy(
                    tp_hbm.at[idx_v.at[sl]], g_v.at[pl.ds(sl * BW, BW)], sem
                ).wait()
            for sl in range(SB):
                def bc_body(bc, carry2):
                    row = sl * BW + bc * L + lane
                    colbase = par_v[sl, pl.ds(bc * L, L)] * D
                    for d in range(D):
                        vals = plsc.load_gather(g_v, [row, colbase + d])
                        t_v[sl, d, pl.ds(bc * L, L)] = vals
                    return carry2

                lax.fori_loop(0, BW // L, bc_body, 0)
            pltpu.sync_copy(
                t_v, out_hbm.at[pl.ds(sb * SB, SB), :, pl.ds(b0, BW)]
            )
            return carry

        lax.fori_loop(0, n_sb, sb_body, 0)

    tpairs = pack(table.T)
    xT = x.T
    idxh3 = (xT >> 1).reshape(n_sb, SB, B0)
    par3 = (xT & 1).reshape(n_sb, SB, B0)
    out3 = emb(tpairs, idxh3, par3)
    return out3.transpose(2, 0, 1)


# speedup vs baseline: 1.5764x; 1.5764x over previous
import functools
import math

import jax
import jax.numpy as jnp
from jax import lax
from jax.experimental import pallas as pl
from jax.experimental.pallas import tpu as pltpu
from jax.experimental.pallas import tpu_sc as plsc


def kernel(x, table):
    B0, S = x.shape
    V, D = table.shape
    W = 2 * D
    scale = math.sqrt(D)

    info = plsc.get_sparse_core_info()
    NC, NS, L = info.num_cores, info.num_subcores, info.num_lanes
    NW = NC * NS

    VB = 128
    n_full = V // VB
    tail = V - n_full * VB
    per_w = n_full // NW
    n_extra = n_full - per_w * NW

    mesh = plsc.VectorSubcoreMesh(core_axis_name="c", subcore_axis_name="s")
    cparams = pltpu.CompilerParams(
        use_tc_tiling_on_sc=True, needs_layout_passes=False
    )

    @functools.partial(
        pl.kernel,
        mesh=mesh,
        out_type=jax.ShapeDtypeStruct((V // 2, W), jnp.float32),
        scratch_types=[
            pltpu.VMEM((D, VB), jnp.float32),
            pltpu.VMEM((VB // 2, W), jnp.float32),
            pltpu.VMEM((D, D), jnp.float32),
        ],
        compiler_params=cparams,
    )
    def pack(tt_hbm, out_hbm, src_v, tr_v, tail_v):
        wid = lax.axis_index("s") * NC + lax.axis_index("c")
        lane = lax.iota(jnp.int32, L)

        rowsel = [dc * L + lane for dc in range(D // L)]

        def do_block(blk):
            v0 = blk * VB
            pltpu.sync_copy(tt_hbm.at[:, pl.ds(v0, VB)], src_v)

            @plsc.parallel_loop(0, VB, unroll=8)
            def _(vl):
                for dc in range(D // L):
                    vals = plsc.load_gather(
                        src_v, [rowsel[dc], jnp.full((L,), vl, jnp.int32)]
                    )
                    tr_v[vl // 2, pl.ds((vl % 2) * D + dc * L, L)] = vals * scale

            pltpu.sync_copy(tr_v, out_hbm.at[pl.ds(blk * (VB // 2), VB // 2)])

        def blk_body(c, carry):
            do_block(wid * per_w + c)
            return carry

        lax.fori_loop(0, per_w, blk_body, 0)

        @pl.when(wid < n_extra)
        def _():
            do_block(NW * per_w + wid)

        @pl.when(wid == n_extra)
        def _():
            v0 = n_full * VB

            def d_body(d, carry):
                pltpu.sync_copy(tt_hbm.at[d, pl.ds(v0, tail)], tail_v.at[d])
                return carry

            lax.fori_loop(0, D, d_body, 0)

            @plsc.parallel_loop(0, tail, unroll=8)
            def _(vl):
                for dc in range(D // L):
                    vals = plsc.load_gather(
                        tail_v, [rowsel[dc], jnp.full((L,), vl, jnp.int32)]
                    )
                    tr_v[vl // 2, pl.ds((vl % 2) * D + dc * L, L)] = vals * scale
            pltpu.sync_copy(
                tr_v.at[pl.ds(0, tail // 2)],
                out_hbm.at[pl.ds(v0 // 2, tail // 2)],
            )

    SB = 4
    n_sb = S // SB
    BW = B0 // NW
    RG = SB * BW

    @functools.partial(
        pl.kernel,
        mesh=mesh,
        out_type=jax.ShapeDtypeStruct((S, D, B0), jnp.float32),
        scratch_types=[
            pltpu.VMEM((SB, BW), jnp.int32),
            pltpu.VMEM((SB, BW), jnp.int32),
            pltpu.VMEM((RG, W), jnp.float32),
            pltpu.VMEM((SB, D, BW), jnp.float32),
            pltpu.SemaphoreType.DMA,
        ],
        compiler_params=cparams,
    )
    def emb(tp_hbm, idxh_hbm, par_hbm, out_hbm, idx_v, par_v, g_v, t_v, sem):
        wid = lax.axis_index("s") * NC + lax.axis_index("c")
        b0 = wid * BW
        lane = lax.iota(jnp.int32, L)

        def sb_body(sb, carry):
            pltpu.sync_copy(idxh_hbm.at[sb, :, pl.ds(b0, BW)], idx_v)
            pltpu.sync_copy(par_hbm.at[sb, :, pl.ds(b0, BW)], par_v)
            for sl in range(SB):
                pltpu.async_copy(
                    tp_hbm.at[idx_v.at[sl]], g_v.at[pl.ds(sl * BW, BW)], sem
                ).wait()
            for sl in range(SB):
                @plsc.parallel_loop(0, BW // L, unroll=2)
                def _(bc):
                    row = sl * BW + bc * L + lane
                    colbase = par_v[sl, pl.ds(bc * L, L)] * D
                    for d in range(D):
                        vals = plsc.load_gather(g_v, [row, colbase + d])
                        t_v[sl, d, pl.ds(bc * L, L)] = vals
            pltpu.sync_copy(
                t_v, out_hbm.at[pl.ds(sb * SB, SB), :, pl.ds(b0, BW)]
            )
            return carry

        lax.fori_loop(0, n_sb, sb_body, 0)

    tpairs = pack(table.T)
    xT = x.T
    idxh3 = (xT >> 1).reshape(n_sb, SB, B0)
    par3 = (xT & 1).reshape(n_sb, SB, B0)
    out3 = emb(tpairs, idxh3, par3)
    return out3.transpose(2, 0, 1)


# speedup vs baseline: 2.1793x; 1.3825x over previous
import functools
import math

import jax
import jax.numpy as jnp
from jax import lax
from jax.experimental import pallas as pl
from jax.experimental.pallas import tpu as pltpu
from jax.experimental.pallas import tpu_sc as plsc


def kernel(x, table):
    B0, S = x.shape
    V, D = table.shape
    W = 2 * D
    scale = math.sqrt(D)

    info = plsc.get_sparse_core_info()
    NC, NS, L = info.num_cores, info.num_subcores, info.num_lanes
    NW = NC * NS

    mesh = plsc.VectorSubcoreMesh(core_axis_name="c", subcore_axis_name="s")
    cparams = pltpu.CompilerParams(
        use_tc_tiling_on_sc=True, needs_layout_passes=False
    )

    VB = 256
    n_full = V // VB
    tail = V - n_full * VB
    per_w = n_full // NW
    n_extra = n_full - per_w * NW

    @functools.partial(
        pl.kernel,
        mesh=mesh,
        out_type=jax.ShapeDtypeStruct((V // 2, W), jnp.float32),
        scratch_types=[
            pltpu.VMEM((2, D, VB), jnp.float32),
            pltpu.VMEM((2, VB // 2, W), jnp.float32),
            pltpu.VMEM((D, D), jnp.float32),
            pltpu.SemaphoreType.DMA,
            pltpu.SemaphoreType.DMA,
        ],
        compiler_params=cparams,
    )
    def pack(tt_hbm, out_hbm, src_v, tr_v, tail_v, sin, sout):
        wid = lax.axis_index("s") * NC + lax.axis_index("c")
        lane = lax.iota(jnp.int32, L)
        rowsel = [dc * L + lane for dc in range(D // L)]
        base = wid * per_w

        def issue_in(c, p):
            pltpu.async_copy(
                tt_hbm.at[:, pl.ds((base + c) * VB, VB)], src_v.at[p], sin
            )

        def wait_in(p):
            pltpu.make_async_copy(
                tt_hbm.at[:, pl.ds(0, VB)], src_v.at[p], sin
            ).wait()

        def issue_out(c, p):
            pltpu.async_copy(
                tr_v.at[p],
                out_hbm.at[pl.ds((base + c) * (VB // 2), VB // 2)],
                sout,
            )

        def wait_out(p):
            pltpu.make_async_copy(
                tr_v.at[p], out_hbm.at[pl.ds(0, VB // 2)], sout
            ).wait()

        def transpose_block(sv, tv, nv):
            @plsc.parallel_loop(0, nv, unroll=8)
            def _(vl):
                for dc in range(D // L):
                    vals = plsc.load_gather(
                        sv, [rowsel[dc], jnp.full((L,), vl, jnp.int32)]
                    )
                    tv[vl // 2, pl.ds((vl % 2) * D + dc * L, L)] = vals * scale

        issue_in(0, 0)
        issue_in(1, 1)

        def cb_body(cb, carry):
            for b in range(2):
                c = cb * 2 + b
                pl_when = pl.when(c >= 2)(lambda: wait_out(b))
                wait_in(b)
                transpose_block(src_v.at[b], tr_v.at[b], VB)
                issue_out(c, b)

                @pl.when(c + 2 < per_w)
                def _():
                    issue_in(c + 2, b)

            return carry

        lax.fori_loop(0, per_w // 2, cb_body, 0)
        wait_out(0)
        wait_out(1)

        @pl.when(wid < n_extra)
        def _():
            blk = n_full - n_extra + wid
            pltpu.async_copy(
                tt_hbm.at[:, pl.ds(blk * VB, VB)], src_v.at[0], sin
            ).wait()
            transpose_block(src_v.at[0], tr_v.at[0], VB)
            pltpu.async_copy(
                tr_v.at[0], out_hbm.at[pl.ds(blk * (VB // 2), VB // 2)], sout
            ).wait()

        @pl.when(wid == n_extra)
        def _():
            v0 = n_full * VB

            def d_body(d, carry):
                pltpu.sync_copy(tt_hbm.at[d, pl.ds(v0, tail)], tail_v.at[d])
                return carry

            lax.fori_loop(0, D, d_body, 0)

            @plsc.parallel_loop(0, tail, unroll=8)
            def _(vl):
                for dc in range(D // L):
                    vals = plsc.load_gather(
                        tail_v, [rowsel[dc], jnp.full((L,), vl, jnp.int32)]
                    )
                    tr_v[0, vl // 2, pl.ds((vl % 2) * D + dc * L, L)] = (
                        vals * scale
                    )

            pltpu.async_copy(
                tr_v.at[0, pl.ds(0, tail // 2)],
                out_hbm.at[pl.ds(v0 // 2, tail // 2)],
                sout,
            ).wait()

    SB = 2
    n_sb = S // SB
    BW = B0 // NW
    RG = SB * BW

    @functools.partial(
        pl.kernel,
        mesh=mesh,
        out_type=jax.ShapeDtypeStruct((S, D, B0), jnp.float32),
        scratch_types=[
            pltpu.VMEM((2, SB, 2, BW), jnp.int32),
            pltpu.VMEM((2, RG, W), jnp.float32),
            pltpu.VMEM((2, SB, D, BW), jnp.float32),
            pltpu.SemaphoreType.DMA,
            pltpu.SemaphoreType.DMA,
            pltpu.SemaphoreType.DMA,
        ],
        compiler_params=cparams,
    )
    def emb(tp_hbm, comb_hbm, out_hbm, cb_v, g_v, t_v, s_cb, s_g, s_out):
        wid = lax.axis_index("s") * NC + lax.axis_index("c")
        b0 = wid * BW
        lane = lax.iota(jnp.int32, L)

        def issue_cb(c, p):
            pltpu.async_copy(
                comb_hbm.at[c, :, :, pl.ds(b0, BW)], cb_v.at[p], s_cb
            )

        def wait_cb(p):
            pltpu.make_async_copy(
                comb_hbm.at[0, :, :, pl.ds(b0, BW)], cb_v.at[p], s_cb
            ).wait()

        def issue_g(p):
            for sl in range(SB):
                pltpu.async_copy(
                    tp_hbm.at[cb_v.at[p, sl, 0]],
                    g_v.at[p, pl.ds(sl * BW, BW)],
                    s_g,
                )

        def wait_g(p):
            for sl in range(SB):
                pltpu.make_async_copy(
                    tp_hbm.at[cb_v.at[p, sl, 0]],
                    g_v.at[p, pl.ds(sl * BW, BW)],
                    s_g,
                ).wait()

        def issue_out(c, p):
            pltpu.async_copy(
                t_v.at[p],
                out_hbm.at[pl.ds(c * SB, SB), :, pl.ds(b0, BW)],
                s_out,
            )

        def wait_out(p):
            pltpu.make_async_copy(
                t_v.at[p], out_hbm.at[pl.ds(0, SB), :, pl.ds(b0, BW)], s_out
            ).wait()

        def compute(p):
            for sl in range(SB):
                @plsc.parallel_loop(0, BW // L, unroll=2)
                def _(bc):
                    row = sl * BW + bc * L + lane
                    colbase = cb_v[p, sl, 1, pl.ds(bc * L, L)] * D
                    for d in range(D):
                        vals = plsc.load_gather(g_v.at[p], [row, colbase + d])
                        t_v[p, sl, d, pl.ds(bc * L, L)] = vals

        issue_cb(0, 0)
        issue_cb(1, 1)
        wait_cb(0)
        issue_g(0)

        def c_body(c, carry):
            p = lax.rem(c, 2)
            for b in range(2):
                @pl.when(p == b)
                def _():
                    q = 1 - b

                    @pl.when(c + 1 < n_sb)
                    def _():
                        wait_cb(q)
                        issue_g(q)

                    wait_g(b)

                    @pl.when(c >= 2)
                    def _():
                        wait_out(b)

                    compute(b)
                    issue_out(c, b)

                    @pl.when(c + 2 < n_sb)
                    def _():
                        issue_cb(c + 2, b)

            return carry

        lax.fori_loop(0, n_sb, c_body, 0)
        wait_out(0)
        wait_out(1)

    tpairs = pack(table.T)
    xT = x.T
    comb3 = jnp.stack([xT >> 1, xT & 1], axis=1).reshape(n_sb, SB, 2, B0)
    out3 = emb(tpairs, comb3)
    return out3.transpose(2, 0, 1)
